# baseline (speedup 1.0000x reference)
import functools

import jax
import jax.numpy as jnp
from jax.experimental import pallas as pl

_B, _N, _T, _C, _S = 8, 2048, 12, 8, 16
_EPS = 1e-5


def _stats_kernel(x_ref, gamma_ref, beta_ref, scale_ref, shift_ref):
    x = x_ref[...]
    denom = x.shape[0] * x.shape[2]
    s1 = jnp.sum(x, axis=(0, 2), keepdims=True)[0]
    s2 = jnp.sum(x * x, axis=(0, 2), keepdims=True)[0]
    mean = s1 / denom
    var = s2 / denom - mean * mean
    scale = gamma_ref[...] * jax.lax.rsqrt(var + _EPS)
    scale_ref[...] = scale
    shift_ref[...] = beta_ref[...] - mean * scale


def _main_kernel(x_ref, a_ref, scale_ref, shift_ref,
                 w1_ref, b1_ref, w2_ref, b2_ref, out_ref):
    bn = x_ref[0] * scale_ref[...] + shift_ref[...]
    y1 = jnp.dot(a_ref[...], bn, preferred_element_type=jnp.float32)
    h = jnp.maximum(
        jnp.dot(y1, w1_ref[...], preferred_element_type=jnp.float32)
        + b1_ref[...], 0.0)
    y2 = jnp.dot(a_ref[...], h, preferred_element_type=jnp.float32)
    o = jnp.dot(y2, w2_ref[...], preferred_element_type=jnp.float32) \
        + b2_ref[...]
    out_ref[0] = jax.nn.sigmoid(o)


@jax.jit
def kernel(X, A, gamma, beta, W1, b1, W2, b2):
    B, N, T, C = X.shape
    S = W1.shape[1]
    X3 = X.reshape(B, N, T * C)

    scale, shift = pl.pallas_call(
        _stats_kernel,
        out_shape=(jax.ShapeDtypeStruct((N, 1), jnp.float32),
                   jax.ShapeDtypeStruct((N, 1), jnp.float32)),
    )(X3, gamma.reshape(N, 1), beta.reshape(N, 1))

    eye_t = jnp.eye(T, dtype=jnp.float32)
    W1e = jnp.kron(eye_t, W1)
    W2e = jnp.kron(eye_t, W2)
    b1e = jnp.tile(b1, T).reshape(1, T * S)
    b2e = jnp.tile(b2, T).reshape(1, T * S)

    out = pl.pallas_call(
        _main_kernel,
        grid=(B,),
        in_specs=[
            pl.BlockSpec((1, N, T * C), lambda b: (b, 0, 0)),
            pl.BlockSpec((N, N), lambda b: (0, 0)),
            pl.BlockSpec((N, 1), lambda b: (0, 0)),
            pl.BlockSpec((N, 1), lambda b: (0, 0)),
            pl.BlockSpec((T * C, T * S), lambda b: (0, 0)),
            pl.BlockSpec((1, T * S), lambda b: (0, 0)),
            pl.BlockSpec((T * S, T * S), lambda b: (0, 0)),
            pl.BlockSpec((1, T * S), lambda b: (0, 0)),
        ],
        out_specs=pl.BlockSpec((1, N, T * S), lambda b: (b, 0, 0)),
        out_shape=jax.ShapeDtypeStruct((B, N, T * S), jnp.float32),
    )(X3, A, scale, shift, W1e, b1e, W2e, b2e)

    return out.reshape(B, N, T, S)
